# baseline (speedup 1.0000x reference)
import functools

import jax
import jax.numpy as jnp
from jax import lax
from jax.experimental import pallas as pl
from jax.experimental.pallas import tpu as pltpu
from jax.experimental.pallas import tpu_sc as plsc

_VOCAB = 1024
_COMM = 64
_NUM_COMMS = 8
_BETA = 0.25

_ROWS_BLK = 512

_NC = 2
_NS = 16
_NW = _NC * _NS
_GATHER_W = 128


_LANES = 128


def _vq_tc_body(x_ref, w_ref, b_ref, cb_ref, idx_ref, dsum_ref, cbt_ref):
    first = (pl.program_id(0) == 0) & (pl.program_id(1) == 0)

    @pl.when(first)
    def _precompute():
        cbt = jnp.transpose(cb_ref[...], (1, 0))
        cbt_ref[0:_COMM, :] = -2.0 * cbt
        cbt_ref[_COMM:_COMM + 1, :] = jnp.sum(cbt * cbt, axis=0,
                                              keepdims=True)

    x = x_ref[...]
    cbn = cbt_ref[0:_COMM, :]
    cnorm = cbt_ref[_COMM:_COMM + 1, :]
    wg = w_ref[0]
    bg = b_ref[0, 0, :]
    fg = lax.dot_general(x, wg, (((1,), (1,)), ((), ())),
                         preferred_element_type=jnp.float32)
    fg = fg + bg[None, :]
    cross = lax.dot_general(fg, cbn, (((1,), (0,)), ((), ())),
                            preferred_element_type=jnp.float32)
    scores = cnorm + cross
    lane = lax.broadcasted_iota(jnp.int32, (_ROWS_BLK, _LANES),
                                1).astype(jnp.float32)
    cur_val = scores[:, 0:_LANES]
    cur_idx = lane
    for k in range(1, _VOCAB // _LANES):
        sk = scores[:, k * _LANES:(k + 1) * _LANES]
        lt = sk < cur_val
        cur_val = jnp.where(lt, sk, cur_val)
        cur_idx = jnp.where(lt, float(k * _LANES) + lane, cur_idx)
    minval = jnp.min(cur_val, axis=1, keepdims=True)
    masked = jnp.where(cur_val == minval, cur_idx, 0.0)
    ones = jnp.ones((_LANES, 1), jnp.float32)
    idxf = lax.dot_general(masked, ones, (((1,), (0,)), ((), ())),
                           precision=lax.Precision.HIGHEST,
                           preferred_element_type=jnp.float32)
    idx = jnp.minimum(idxf[:, 0], float(_VOCAB - 1)).astype(jnp.int32)
    idx_ref[0, 0, 0, :] = idx
    fnorm = jnp.sum(fg * fg, axis=1)
    partial = jnp.sum(minval[:, 0] + fnorm)

    @pl.when(first)
    def _init():
        dsum_ref[...] = jnp.zeros_like(dsum_ref)

    dsum_ref[...] += partial.reshape(1, 1)


def _vq_argmin(xf, W3, b3, codebook):
    rows = xf.shape[0]
    grid = rows // _ROWS_BLK
    return pl.pallas_call(
        _vq_tc_body,
        grid=(grid, _NUM_COMMS),
        in_specs=[
            pl.BlockSpec((_ROWS_BLK, xf.shape[1]), lambda i, j: (i, 0)),
            pl.BlockSpec((1, _COMM, W3.shape[2]), lambda i, j: (j, 0, 0)),
            pl.BlockSpec((1, 1, _COMM), lambda i, j: (j, 0, 0)),
            pl.BlockSpec(codebook.shape, lambda i, j: (0, 0)),
        ],
        out_specs=[
            pl.BlockSpec((1, 1, 1, _ROWS_BLK), lambda i, j: (i, j, 0, 0)),
            pl.BlockSpec((1, 1), lambda i, j: (0, 0)),
        ],
        out_shape=[
            jax.ShapeDtypeStruct((grid, _NUM_COMMS, 1, _ROWS_BLK), jnp.int32),
            jax.ShapeDtypeStruct((1, 1), jnp.float32),
        ],
        scratch_shapes=[
            pltpu.VMEM((_COMM + 1, _VOCAB), jnp.float32),
        ],
    )(xf, W3, b3, codebook)


_CHUNK_ELS = _GATHER_W * _COMM


def _sc_gather_body(cb_hbm, idx_hbm, out_hbm, cb_v, idx_v, buf, sem):
    wid = lax.axis_index("s") * _NC + lax.axis_index("c")
    per_w = idx_hbm.shape[0] // _NW
    rbase = wid * per_w
    pltpu.sync_copy(cb_hbm, cb_v)
    pltpu.sync_copy(idx_hbm.at[pl.ds(rbase, per_w)], idx_v)
    lane64 = lax.broadcasted_iota(jnp.int32, (16,), 0) * _COMM

    def chunk_body(g, carry):
        @plsc.parallel_loop(0, _GATHER_W // 16)
        def inner(j):
            tok = idx_v[g, pl.ds(j * 16, 16)]
            src = tok * _COMM
            dstbase = lane64 + j * (16 * _COMM)
            for c in range(_COMM):
                vals = plsc.load_gather(cb_v, [src + c])
                plsc.store_scatter(buf, [dstbase + c], vals)

        pltpu.sync_copy(buf,
                        out_hbm.at[pl.ds((rbase + g) * _CHUNK_ELS,
                                         _CHUNK_ELS)])
        return carry

    lax.fori_loop(0, per_w, chunk_body, 0)


def _sc_gather(codebook, idx2):
    tokens = idx2.shape[0] * idx2.shape[1]
    per_w = idx2.shape[0] // _NW
    fn = functools.partial(
        pl.kernel,
        out_type=jax.ShapeDtypeStruct((tokens * _COMM,), jnp.float32),
        mesh=plsc.VectorSubcoreMesh(core_axis_name="c", subcore_axis_name="s"),
        compiler_params=pltpu.CompilerParams(use_tc_tiling_on_sc=False,
                                             needs_layout_passes=False),
        scratch_types=[
            pltpu.VMEM((_VOCAB * _COMM,), jnp.float32),
            pltpu.VMEM((per_w, _GATHER_W), jnp.int32),
            pltpu.VMEM((_CHUNK_ELS,), jnp.float32),
            pltpu.SemaphoreType.DMA,
        ],
    )(_sc_gather_body)
    return fn(codebook.reshape(-1), idx2)


def kernel(x, W, b, codebook):
    B, T, N, H = x.shape
    xf = x.reshape(B * T * N, H)
    W3 = W.reshape(_NUM_COMMS, _COMM, H)
    b3 = b.reshape(_NUM_COMMS, 1, _COMM)
    idx4, dsum = _vq_argmin(xf, W3, b3, codebook)
    tokens = B * T * N * _NUM_COMMS
    idx2 = jnp.transpose(idx4[:, :, 0, :], (0, 2, 1)).reshape(
        tokens // _GATHER_W, _GATHER_W)
    qflat = _sc_gather(codebook, idx2)
    vq_loss = (1.0 + _BETA) * dsum[0, 0] / (tokens * _COMM)
    comm_output = qflat.reshape(B, N, _NUM_COMMS, _COMM)
    comm_log_probs = jnp.zeros((B * T, N), dtype=jnp.float32)
    return (comm_output, comm_output, comm_log_probs, vq_loss)


# speedup vs baseline: 1.1483x; 1.1483x over previous
import functools

import jax
import jax.numpy as jnp
from jax import lax
from jax.experimental import pallas as pl
from jax.experimental.pallas import tpu as pltpu
from jax.experimental.pallas import tpu_sc as plsc

_VOCAB = 1024
_COMM = 64
_NUM_COMMS = 8
_BETA = 0.25

_ROWS_BLK = 1024

_NC = 2
_NS = 16
_NW = _NC * _NS
_GATHER_W = 128


_LANES = 128


def _vq_tc_body(x_ref, w_ref, b_ref, cb_ref, idx_ref, dsum_ref, cbt_ref):
    first = (pl.program_id(0) == 0) & (pl.program_id(1) == 0)

    @pl.when(first)
    def _precompute():
        cbt = jnp.transpose(cb_ref[...], (1, 0))
        cbt_ref[0:_COMM, :] = -2.0 * cbt
        cbt_ref[_COMM:_COMM + 1, :] = jnp.sum(cbt * cbt, axis=0,
                                              keepdims=True)

    x = x_ref[...]
    cbn = cbt_ref[0:_COMM, :]
    cnorm = cbt_ref[_COMM:_COMM + 1, :]
    wg = w_ref[0]
    bg = b_ref[0, 0, :]
    fg = lax.dot_general(x, wg, (((1,), (1,)), ((), ())),
                         preferred_element_type=jnp.float32)
    fg = fg + bg[None, :]
    cross = lax.dot_general(fg, cbn, (((1,), (0,)), ((), ())),
                            preferred_element_type=jnp.float32)
    scores = cnorm + cross
    lane = lax.broadcasted_iota(jnp.int32, (_ROWS_BLK, _LANES),
                                1).astype(jnp.float32)
    cur_val = scores[:, 0:_LANES]
    cur_idx = lane
    for k in range(1, _VOCAB // _LANES):
        sk = scores[:, k * _LANES:(k + 1) * _LANES]
        lt = sk < cur_val
        cur_val = jnp.where(lt, sk, cur_val)
        cur_idx = jnp.where(lt, float(k * _LANES) + lane, cur_idx)
    minval = jnp.min(cur_val, axis=1, keepdims=True)
    masked = jnp.where(cur_val == minval, cur_idx, 0.0)
    ones = jnp.ones((_LANES, 1), jnp.float32)
    idxf = lax.dot_general(masked, ones, (((1,), (0,)), ((), ())),
                           precision=lax.Precision.HIGHEST,
                           preferred_element_type=jnp.float32)
    idx = jnp.minimum(idxf[:, 0], float(_VOCAB - 1)).astype(jnp.int32)
    idx_ref[0, 0, 0, :] = idx
    fnorm = jnp.sum(fg * fg, axis=1)
    partial = jnp.sum(minval[:, 0] + fnorm)

    @pl.when(first)
    def _init():
        dsum_ref[...] = jnp.zeros_like(dsum_ref)

    dsum_ref[...] += partial.reshape(1, 1)


def _vq_argmin(xf, W3, b3, codebook):
    rows = xf.shape[0]
    grid = rows // _ROWS_BLK
    return pl.pallas_call(
        _vq_tc_body,
        grid=(grid, _NUM_COMMS),
        in_specs=[
            pl.BlockSpec((_ROWS_BLK, xf.shape[1]), lambda i, j: (i, 0)),
            pl.BlockSpec((1, _COMM, W3.shape[2]), lambda i, j: (j, 0, 0)),
            pl.BlockSpec((1, 1, _COMM), lambda i, j: (j, 0, 0)),
            pl.BlockSpec(codebook.shape, lambda i, j: (0, 0)),
        ],
        out_specs=[
            pl.BlockSpec((1, 1, 1, _ROWS_BLK), lambda i, j: (i, j, 0, 0)),
            pl.BlockSpec((1, 1), lambda i, j: (0, 0)),
        ],
        out_shape=[
            jax.ShapeDtypeStruct((grid, _NUM_COMMS, 1, _ROWS_BLK), jnp.int32),
            jax.ShapeDtypeStruct((1, 1), jnp.float32),
        ],
        scratch_shapes=[
            pltpu.VMEM((_COMM + 1, _VOCAB), jnp.float32),
        ],
    )(xf, W3, b3, codebook)


_CHUNK_ELS = _GATHER_W * _COMM


def _sc_gather_body(cb_hbm, idx_hbm, out_hbm, cb_v, idx_v, buf, sem):
    wid = lax.axis_index("s") * _NC + lax.axis_index("c")
    per_w = idx_hbm.shape[0] // _NW
    rbase = wid * per_w
    pltpu.sync_copy(cb_hbm, cb_v)
    pltpu.sync_copy(idx_hbm.at[pl.ds(rbase, per_w)], idx_v)
    lane64 = lax.broadcasted_iota(jnp.int32, (16,), 0) * _COMM

    def chunk_body(g, carry):
        @plsc.parallel_loop(0, _GATHER_W // 16 * _COMM, unroll=8)
        def inner(t):
            j = t // _COMM
            c = t % _COMM
            tok = idx_v[g, pl.ds(j * 16, 16)]
            vals = plsc.load_gather(cb_v, [tok * _COMM + c])
            plsc.store_scatter(buf, [lane64 + (j * 16 * _COMM + c)], vals)

        pltpu.sync_copy(buf,
                        out_hbm.at[pl.ds((rbase + g) * _CHUNK_ELS,
                                         _CHUNK_ELS)])
        return carry

    lax.fori_loop(0, per_w, chunk_body, 0)


def _sc_gather(codebook, idx2):
    tokens = idx2.shape[0] * idx2.shape[1]
    per_w = idx2.shape[0] // _NW
    fn = functools.partial(
        pl.kernel,
        out_type=jax.ShapeDtypeStruct((tokens * _COMM,), jnp.float32),
        mesh=plsc.VectorSubcoreMesh(core_axis_name="c", subcore_axis_name="s"),
        compiler_params=pltpu.CompilerParams(use_tc_tiling_on_sc=False,
                                             needs_layout_passes=False),
        scratch_types=[
            pltpu.VMEM((_VOCAB * _COMM,), jnp.float32),
            pltpu.VMEM((per_w, _GATHER_W), jnp.int32),
            pltpu.VMEM((_CHUNK_ELS,), jnp.float32),
            pltpu.SemaphoreType.DMA,
        ],
    )(_sc_gather_body)
    return fn(codebook.reshape(-1), idx2)


def kernel(x, W, b, codebook):
    B, T, N, H = x.shape
    xf = x.reshape(B * T * N, H)
    W3 = W.reshape(_NUM_COMMS, _COMM, H)
    b3 = b.reshape(_NUM_COMMS, 1, _COMM)
    idx4, dsum = _vq_argmin(xf, W3, b3, codebook)
    tokens = B * T * N * _NUM_COMMS
    idx2 = jnp.transpose(idx4[:, :, 0, :], (0, 2, 1)).reshape(
        tokens // _GATHER_W, _GATHER_W)
    qflat = _sc_gather(codebook, idx2)
    vq_loss = (1.0 + _BETA) * dsum[0, 0] / (tokens * _COMM)
    comm_output = qflat.reshape(B, N, _NUM_COMMS, _COMM)
    comm_log_probs = jnp.zeros((B * T, N), dtype=jnp.float32)
    return (comm_output, comm_output, comm_log_probs, vq_loss)


# speedup vs baseline: 1.2056x; 1.0499x over previous
import functools

import jax
import jax.numpy as jnp
from jax import lax
from jax.experimental import pallas as pl
from jax.experimental.pallas import tpu as pltpu
from jax.experimental.pallas import tpu_sc as plsc

_VOCAB = 1024
_COMM = 64
_NUM_COMMS = 8
_BETA = 0.25

_ROWS_BLK = 2048

_NC = 2
_NS = 16
_NW = _NC * _NS
_GATHER_W = 128


_LANES = 128


def _vq_tc_body(x_ref, w_ref, b_ref, cb_ref, idx_ref, dsum_ref, cbt_ref):
    first = (pl.program_id(0) == 0) & (pl.program_id(1) == 0)

    @pl.when(first)
    def _precompute():
        cbt = jnp.transpose(cb_ref[...], (1, 0))
        cbt_ref[0:_COMM, :] = -2.0 * cbt
        cbt_ref[_COMM:_COMM + 1, :] = jnp.sum(cbt * cbt, axis=0,
                                              keepdims=True)

    x = x_ref[...]
    cbn = cbt_ref[0:_COMM, :]
    cnorm = cbt_ref[_COMM:_COMM + 1, :]
    wg = w_ref[0]
    bg = b_ref[0, 0, :]
    fg = lax.dot_general(x, wg, (((1,), (1,)), ((), ())),
                         preferred_element_type=jnp.float32)
    fg = fg + bg[None, :]
    cross = lax.dot_general(fg, cbn, (((1,), (0,)), ((), ())),
                            preferred_element_type=jnp.float32)
    scores = cnorm + cross
    lane = lax.broadcasted_iota(jnp.int32, (_ROWS_BLK, _LANES),
                                1).astype(jnp.float32)
    cur_val = scores[:, 0:_LANES]
    cur_idx = lane
    for k in range(1, _VOCAB // _LANES):
        sk = scores[:, k * _LANES:(k + 1) * _LANES]
        lt = sk < cur_val
        cur_val = jnp.where(lt, sk, cur_val)
        cur_idx = jnp.where(lt, float(k * _LANES) + lane, cur_idx)
    minval = jnp.min(cur_val, axis=1, keepdims=True)
    masked = jnp.where(cur_val == minval, cur_idx, 0.0)
    ones = jnp.ones((_LANES, 1), jnp.float32)
    idxf = lax.dot_general(masked, ones, (((1,), (0,)), ((), ())),
                           precision=lax.Precision.HIGHEST,
                           preferred_element_type=jnp.float32)
    idx = jnp.minimum(idxf[:, 0], float(_VOCAB - 1)).astype(jnp.int32)
    idx_ref[0, 0, 0, :] = idx
    fnorm = jnp.sum(fg * fg, axis=1)
    partial = jnp.sum(minval[:, 0] + fnorm)

    @pl.when(first)
    def _init():
        dsum_ref[...] = jnp.zeros_like(dsum_ref)

    dsum_ref[...] += partial.reshape(1, 1)


def _vq_argmin(xf, W3, b3, codebook):
    rows = xf.shape[0]
    grid = rows // _ROWS_BLK
    return pl.pallas_call(
        _vq_tc_body,
        grid=(grid, _NUM_COMMS),
        in_specs=[
            pl.BlockSpec((_ROWS_BLK, xf.shape[1]), lambda i, j: (i, 0)),
            pl.BlockSpec((1, _COMM, W3.shape[2]), lambda i, j: (j, 0, 0)),
            pl.BlockSpec((1, 1, _COMM), lambda i, j: (j, 0, 0)),
            pl.BlockSpec(codebook.shape, lambda i, j: (0, 0)),
        ],
        out_specs=[
            pl.BlockSpec((1, 1, 1, _ROWS_BLK), lambda i, j: (i, j, 0, 0)),
            pl.BlockSpec((1, 1), lambda i, j: (0, 0)),
        ],
        out_shape=[
            jax.ShapeDtypeStruct((grid, _NUM_COMMS, 1, _ROWS_BLK), jnp.int32),
            jax.ShapeDtypeStruct((1, 1), jnp.float32),
        ],
        scratch_shapes=[
            pltpu.VMEM((_COMM + 1, _VOCAB), jnp.float32),
        ],
    )(xf, W3, b3, codebook)


_CHUNK_ELS = _GATHER_W * _COMM


def _sc_gather_body(cb_hbm, idx_hbm, out_hbm, cb_v, idx_v, buf, sem):
    wid = lax.axis_index("s") * _NC + lax.axis_index("c")
    per_w = idx_hbm.shape[0] // _NW
    rbase = wid * per_w
    pltpu.sync_copy(cb_hbm, cb_v)
    pltpu.sync_copy(idx_hbm.at[pl.ds(rbase, per_w)], idx_v)
    lane64 = lax.broadcasted_iota(jnp.int32, (16,), 0) * _COMM

    def chunk_body(g, carry):
        @plsc.parallel_loop(0, _GATHER_W // 16 * _COMM, unroll=8)
        def inner(t):
            j = t // _COMM
            c = t % _COMM
            tok = idx_v[g, pl.ds(j * 16, 16)]
            vals = plsc.load_gather(cb_v, [tok * _COMM + c])
            plsc.store_scatter(buf, [lane64 + (j * 16 * _COMM + c)], vals)

        pltpu.sync_copy(buf,
                        out_hbm.at[pl.ds((rbase + g) * _CHUNK_ELS,
                                         _CHUNK_ELS)])
        return carry

    lax.fori_loop(0, per_w, chunk_body, 0)


def _sc_gather(codebook, idx2):
    tokens = idx2.shape[0] * idx2.shape[1]
    per_w = idx2.shape[0] // _NW
    fn = functools.partial(
        pl.kernel,
        out_type=jax.ShapeDtypeStruct((tokens * _COMM,), jnp.float32),
        mesh=plsc.VectorSubcoreMesh(core_axis_name="c", subcore_axis_name="s"),
        compiler_params=pltpu.CompilerParams(use_tc_tiling_on_sc=False,
                                             needs_layout_passes=False),
        scratch_types=[
            pltpu.VMEM((_VOCAB * _COMM,), jnp.float32),
            pltpu.VMEM((per_w, _GATHER_W), jnp.int32),
            pltpu.VMEM((_CHUNK_ELS,), jnp.float32),
            pltpu.SemaphoreType.DMA,
        ],
    )(_sc_gather_body)
    return fn(codebook.reshape(-1), idx2)


def kernel(x, W, b, codebook):
    B, T, N, H = x.shape
    xf = x.reshape(B * T * N, H)
    W3 = W.reshape(_NUM_COMMS, _COMM, H)
    b3 = b.reshape(_NUM_COMMS, 1, _COMM)
    idx4, dsum = _vq_argmin(xf, W3, b3, codebook)
    tokens = B * T * N * _NUM_COMMS
    idx2 = jnp.transpose(idx4[:, :, 0, :], (0, 2, 1)).reshape(
        tokens // _GATHER_W, _GATHER_W)
    qflat = _sc_gather(codebook, idx2)
    vq_loss = (1.0 + _BETA) * dsum[0, 0] / (tokens * _COMM)
    comm_output = qflat.reshape(B, N, _NUM_COMMS, _COMM)
    comm_log_probs = jnp.zeros((B * T, N), dtype=jnp.float32)
    return (comm_output, comm_output, comm_log_probs, vq_loss)


# speedup vs baseline: 1.2178x; 1.0102x over previous
import functools

import jax
import jax.numpy as jnp
from jax import lax
from jax.experimental import pallas as pl
from jax.experimental.pallas import tpu as pltpu
from jax.experimental.pallas import tpu_sc as plsc

_VOCAB = 1024
_COMM = 64
_NUM_COMMS = 8
_BETA = 0.25

_ROWS_BLK = 4096

_NC = 2
_NS = 16
_NW = _NC * _NS
_GATHER_W = 128


_LANES = 128


def _vq_tc_body(x_ref, w_ref, b_ref, cb_ref, idx_ref, dsum_ref, cbt_ref):
    first = (pl.program_id(0) == 0) & (pl.program_id(1) == 0)

    @pl.when(first)
    def _precompute():
        cbt = jnp.transpose(cb_ref[...], (1, 0))
        cbt_ref[0:_COMM, :] = -2.0 * cbt
        cbt_ref[_COMM:_COMM + 1, :] = jnp.sum(cbt * cbt, axis=0,
                                              keepdims=True)

    x = x_ref[...]
    cbn = cbt_ref[0:_COMM, :]
    cnorm = cbt_ref[_COMM:_COMM + 1, :]
    wg = w_ref[0]
    bg = b_ref[0, 0, :]
    fg = lax.dot_general(x, wg, (((1,), (1,)), ((), ())),
                         preferred_element_type=jnp.float32)
    fg = fg + bg[None, :]
    cross = lax.dot_general(fg, cbn, (((1,), (0,)), ((), ())),
                            preferred_element_type=jnp.float32)
    scores = cnorm + cross
    lane = lax.broadcasted_iota(jnp.int32, (_ROWS_BLK, _LANES),
                                1).astype(jnp.float32)
    cur_val = scores[:, 0:_LANES]
    cur_idx = lane
    for k in range(1, _VOCAB // _LANES):
        sk = scores[:, k * _LANES:(k + 1) * _LANES]
        lt = sk < cur_val
        cur_val = jnp.where(lt, sk, cur_val)
        cur_idx = jnp.where(lt, float(k * _LANES) + lane, cur_idx)
    minval = jnp.min(cur_val, axis=1, keepdims=True)
    masked = jnp.where(cur_val == minval, cur_idx, 0.0)
    ones = jnp.ones((_LANES, 1), jnp.float32)
    idxf = lax.dot_general(masked, ones, (((1,), (0,)), ((), ())),
                           precision=lax.Precision.HIGHEST,
                           preferred_element_type=jnp.float32)
    idx = jnp.minimum(idxf[:, 0], float(_VOCAB - 1)).astype(jnp.int32)
    idx_ref[0, 0, 0, :] = idx
    fnorm = jnp.sum(fg * fg, axis=1)
    partial = jnp.sum(minval[:, 0] + fnorm)

    @pl.when(first)
    def _init():
        dsum_ref[...] = jnp.zeros_like(dsum_ref)

    dsum_ref[...] += partial.reshape(1, 1)


def _vq_argmin(xf, W3, b3, codebook):
    rows = xf.shape[0]
    grid = rows // _ROWS_BLK
    return pl.pallas_call(
        _vq_tc_body,
        grid=(grid, _NUM_COMMS),
        in_specs=[
            pl.BlockSpec((_ROWS_BLK, xf.shape[1]), lambda i, j: (i, 0)),
            pl.BlockSpec((1, _COMM, W3.shape[2]), lambda i, j: (j, 0, 0)),
            pl.BlockSpec((1, 1, _COMM), lambda i, j: (j, 0, 0)),
            pl.BlockSpec(codebook.shape, lambda i, j: (0, 0)),
        ],
        out_specs=[
            pl.BlockSpec((1, 1, 1, _ROWS_BLK), lambda i, j: (i, j, 0, 0)),
            pl.BlockSpec((1, 1), lambda i, j: (0, 0)),
        ],
        out_shape=[
            jax.ShapeDtypeStruct((grid, _NUM_COMMS, 1, _ROWS_BLK), jnp.int32),
            jax.ShapeDtypeStruct((1, 1), jnp.float32),
        ],
        scratch_shapes=[
            pltpu.VMEM((_COMM + 1, _VOCAB), jnp.float32),
        ],
    )(xf, W3, b3, codebook)


_CHUNK_ELS = _GATHER_W * _COMM


def _sc_gather_body(cb_hbm, idx_hbm, out_hbm, cb_v, idx_v, buf, sem):
    wid = lax.axis_index("s") * _NC + lax.axis_index("c")
    per_w = idx_hbm.shape[0] // _NW
    rbase = wid * per_w
    pltpu.sync_copy(cb_hbm, cb_v)
    pltpu.sync_copy(idx_hbm.at[pl.ds(rbase, per_w)], idx_v)
    lane64 = lax.broadcasted_iota(jnp.int32, (16,), 0) * _COMM

    def chunk_body(g, carry):
        @plsc.parallel_loop(0, _GATHER_W // 16 * _COMM, unroll=8)
        def inner(t):
            j = t // _COMM
            c = t % _COMM
            tok = idx_v[g, pl.ds(j * 16, 16)]
            vals = plsc.load_gather(cb_v, [tok * _COMM + c])
            plsc.store_scatter(buf, [lane64 + (j * 16 * _COMM + c)], vals)

        pltpu.sync_copy(buf,
                        out_hbm.at[pl.ds((rbase + g) * _CHUNK_ELS,
                                         _CHUNK_ELS)])
        return carry

    lax.fori_loop(0, per_w, chunk_body, 0)


def _sc_gather(codebook, idx2):
    tokens = idx2.shape[0] * idx2.shape[1]
    per_w = idx2.shape[0] // _NW
    fn = functools.partial(
        pl.kernel,
        out_type=jax.ShapeDtypeStruct((tokens * _COMM,), jnp.float32),
        mesh=plsc.VectorSubcoreMesh(core_axis_name="c", subcore_axis_name="s"),
        compiler_params=pltpu.CompilerParams(use_tc_tiling_on_sc=False,
                                             needs_layout_passes=False),
        scratch_types=[
            pltpu.VMEM((_VOCAB * _COMM,), jnp.float32),
            pltpu.VMEM((per_w, _GATHER_W), jnp.int32),
            pltpu.VMEM((_CHUNK_ELS,), jnp.float32),
            pltpu.SemaphoreType.DMA,
        ],
    )(_sc_gather_body)
    return fn(codebook.reshape(-1), idx2)


def kernel(x, W, b, codebook):
    B, T, N, H = x.shape
    xf = x.reshape(B * T * N, H)
    W3 = W.reshape(_NUM_COMMS, _COMM, H)
    b3 = b.reshape(_NUM_COMMS, 1, _COMM)
    idx4, dsum = _vq_argmin(xf, W3, b3, codebook)
    tokens = B * T * N * _NUM_COMMS
    idx2 = jnp.transpose(idx4[:, :, 0, :], (0, 2, 1)).reshape(
        tokens // _GATHER_W, _GATHER_W)
    qflat = _sc_gather(codebook, idx2)
    vq_loss = (1.0 + _BETA) * dsum[0, 0] / (tokens * _COMM)
    comm_output = qflat.reshape(B, N, _NUM_COMMS, _COMM)
    comm_log_probs = jnp.zeros((B * T, N), dtype=jnp.float32)
    return (comm_output, comm_output, comm_log_probs, vq_loss)
